# baseline (speedup 1.0000x reference)
import functools

import jax
import jax.numpy as jnp
from jax.experimental import pallas as pl

N = 10000
E = 160000
f32 = jnp.float32
bf16 = jnp.bfloat16


def _rbf(a):
    return a.astype(bf16).astype(f32)


def _combine_body(h0, h1, h2, h3, w, b, o, *, din, relu):
    hs = (h0, h1, h2, h3)
    if din == 1:
        acc = hs[0][...] * w[0]
        for k in range(1, 4):
            acc = acc + hs[k][...] * w[k]
    else:
        acc = jnp.dot(hs[0][...].astype(bf16), w[0], preferred_element_type=f32)
        for k in range(1, 4):
            acc = acc + jnp.dot(hs[k][...].astype(bf16), w[k], preferred_element_type=f32)
    acc = acc + b[...]
    if relu:
        acc = jnp.maximum(acc, 0.0)
    o[...] = acc


def _combine(hs, W, b, relu):
    din, dout = W.shape[1], W.shape[2]
    if din == 1:
        Wk = W.astype(f32)
    else:
        Wk = W.astype(bf16)
    BR = 1000
    grid = N // BR
    body = functools.partial(_combine_body, din=din, relu=relu)
    return pl.pallas_call(
        body,
        grid=(grid,),
        in_specs=[pl.BlockSpec((BR, din), lambda i: (i, 0))] * 4
        + [pl.BlockSpec((4, din, dout), lambda i: (0, 0, 0)),
           pl.BlockSpec((1, dout), lambda i: (0, 0))],
        out_specs=pl.BlockSpec((BR, dout), lambda i: (i, 0)),
        out_shape=jax.ShapeDtypeStruct((N, dout), f32),
    )(*hs, Wk, b.reshape(1, dout))


def _mm_body(x, w, o):
    o[...] = jnp.dot(x[...].astype(bf16), w[...], preferred_element_type=f32)


def _mm_bf16(x, Wb):
    n, din = x.shape
    dout = Wb.shape[1]
    BR = 2000
    return pl.pallas_call(
        _mm_body,
        grid=(n // BR,),
        in_specs=[pl.BlockSpec((BR, din), lambda i: (i, 0)),
                  pl.BlockSpec((din, dout), lambda i: (0, 0))],
        out_specs=pl.BlockSpec((BR, dout), lambda i: (i, 0)),
        out_shape=jax.ShapeDtypeStruct((n, dout), f32),
    )(x, Wb)


def _tail_body(s, ea, we, b1, g, beta, w2, b2, o):
    eab = ea[...].astype(bf16).astype(f32)
    h = s[...] + eab[..., None] * we[...].astype(f32) + b1[...]
    h = jnp.maximum(h, 0.0)
    m = jnp.mean(h, axis=-1, keepdims=True)
    v = jnp.mean((h - m) ** 2, axis=-1, keepdims=True)
    h = (h - m) / jnp.sqrt(v + 1e-5) * g[...] + beta[...]
    hb = h.astype(bf16).astype(f32)
    t = jnp.sum(hb * w2[...].astype(f32), axis=-1) + b2[0, 0, 0, 0]
    o[...] = t


def _mlp_tail(S, ea, we, b1, g, beta, W2, b2):
    hid = S.shape[1]
    S4 = S.reshape(10, 125, 128, hid)
    ea3 = ea.reshape(10, 125, 128)
    out = pl.pallas_call(
        _tail_body,
        grid=(10,),
        in_specs=[pl.BlockSpec((1, 125, 128, hid), lambda i: (i, 0, 0, 0)),
                  pl.BlockSpec((1, 125, 128), lambda i: (i, 0, 0)),
                  pl.BlockSpec((1, 1, 1, hid), lambda i: (0, 0, 0, 0)),
                  pl.BlockSpec((1, 1, 1, hid), lambda i: (0, 0, 0, 0)),
                  pl.BlockSpec((1, 1, 1, hid), lambda i: (0, 0, 0, 0)),
                  pl.BlockSpec((1, 1, 1, hid), lambda i: (0, 0, 0, 0)),
                  pl.BlockSpec((1, 1, 1, hid), lambda i: (0, 0, 0, 0)),
                  pl.BlockSpec((1, 1, 1, 1), lambda i: (0, 0, 0, 0))],
        out_specs=pl.BlockSpec((1, 125, 128), lambda i: (i, 0, 0)),
        out_shape=jax.ShapeDtypeStruct((10, 125, 128), f32),
    )(S4, ea3, we.reshape(1, 1, 1, hid).astype(bf16), b1.reshape(1, 1, 1, hid),
      g.reshape(1, 1, 1, hid), beta.reshape(1, 1, 1, hid),
      W2.reshape(1, 1, 1, hid).astype(bf16), b2.reshape(1, 1, 1, 1))
    return out.reshape(E)


def _dis_body(deg, o):
    r = jax.lax.rsqrt(deg[...])
    o[...] = jnp.where(jnp.isinf(r), 0.0, r)


def _dis(deg):
    return pl.pallas_call(
        _dis_body,
        out_shape=jax.ShapeDtypeStruct(deg.shape, f32),
    )(deg)


def _norm_body(a, b, c, o):
    o[...] = a[...] * b[...] * c[...]


def _norm_mul(dr, ew, dc):
    return pl.pallas_call(
        _norm_body,
        out_shape=jax.ShapeDtypeStruct((E,), f32),
    )(dr, ew, dc)


def _final_body(x, o):
    xv = x[...]
    m = jnp.mean(xv)
    d = xv - m
    v = jnp.sum(d * d) / (E - 1)
    o[...] = jnp.abs(d / jnp.sqrt(v))


def _final_norm_abs(ea):
    x = ea.reshape(1250, 128)
    out = pl.pallas_call(
        _final_body,
        out_shape=jax.ShapeDtypeStruct((1250, 128), f32),
    )(x)
    return out.reshape(E, 1)


def _scatter_add(vals, col, width):
    z = jnp.zeros((N, width) if width > 0 else (N,), f32)
    return z.at[col].add(vals)


def _tag_norm(row, col, ew):
    deg = _scatter_add(ew, col, 0)
    dis = _dis(deg)
    return _norm_mul(dis[row], ew, dis[col])


def _tagconv(x, row, col, norm, W, b, relu):
    hs = [x]
    h = x
    for _ in range(3):
        msg = h[row] * norm[:, None]
        h = _scatter_add(msg, col, h.shape[1])
        hs.append(h)
    return _combine(hs, W, b, relu)


def _edge_mlp(xn, row, col, ea, W1, b1, g, beta, W2, b2):
    d = xn.shape[1]
    Wsd = jnp.concatenate([W1[:d], W1[d:2 * d]], axis=1).astype(bf16)
    PQ = _mm_bf16(xn, Wsd)
    hid = W1.shape[1]
    S = PQ[:, :hid][row] + PQ[:, hid:][col]
    return _mlp_tail(S, ea, W1[2 * d], b1, g, beta, W2[:, 0], b2[0])


def kernel(x, edge_index, edge_attr, W1a, b1a, W1b, b1b, W2a, b2a, W2b, b2b, W3a, b3a, W3b, b3b, e1_W1, e1_b1, e1_g, e1_beta, e1_W2, e1_b2, e2_W1, e2_b1, e2_g, e2_beta, e2_W2, e2_b2, e3_W1, e3_b1, e3_g, e3_beta, e3_W2, e3_b2):
    row, col = edge_index[0], edge_index[1]
    n0 = _tag_norm(row, col, edge_attr)
    x1 = _tagconv(x, row, col, n0, W1a, b1a, True)
    x1 = _tagconv(x1, row, col, n0, W1b, b1b, False)
    ea = _edge_mlp(x1, row, col, edge_attr, e1_W1, e1_b1, e1_g, e1_beta, e1_W2, e1_b2)
    n1 = _tag_norm(row, col, ea)
    x2 = _tagconv(x1, row, col, n1, W2a, b2a, True)
    x2 = _tagconv(x2, row, col, n1, W2b, b2b, False)
    ea = _edge_mlp(x2, row, col, ea, e2_W1, e2_b1, e2_g, e2_beta, e2_W2, e2_b2)
    n2 = _tag_norm(row, col, ea)
    x3 = _tagconv(x2, row, col, n2, W3a, b3a, True)
    x3 = _tagconv(x3, row, col, n2, W3b, b3b, False)
    ea = _edge_mlp(x3, row, col, ea, e3_W1, e3_b1, e3_g, e3_beta, e3_W2, e3_b2)
    return _final_norm_abs(ea)
